# Initial kernel scaffold; baseline (speedup 1.0000x reference)
import jax
import jax.numpy as jnp
from jax.experimental import pallas as pl


def kernel(h, edge_index, group_labels, W_self, W_groups):
    raise NotImplementedError("write your pallas kernel here")


# speedup vs baseline: 34.6011x; 34.6011x over previous
import functools

import jax
import jax.numpy as jnp
from jax import lax
from jax.experimental import pallas as pl
from jax.experimental.pallas import tpu as pltpu
from jax.experimental.pallas import tpu_sc as plsc

NC = 2
NS = 16
NW = NC * NS
CHUNK = 125


def _group_transform_body(h_ref, g_ref, wg_ref, out_ref):
    h = h_ref[...]
    g = g_ref[...]
    G = wg_ref.shape[0]
    acc = None
    for gi in range(G):
        hm = jnp.where(g == gi, h, 0.0)
        r = lax.dot_general(hm, wg_ref[gi], (((1,), (1,)), ((), ())),
                            preferred_element_type=jnp.float32)
        acc = r if acc is None else acc + r
    out_ref[...] = acc


def _final_body(h_ref, w_ref, p_ref, out_ref):
    hs = lax.dot_general(h_ref[...], w_ref[...], (((1,), (1,)), ((), ())),
                         preferred_element_type=jnp.float32)
    out_ref[...] = hs + p_ref[0] + p_ref[1]


def _make_sc_scatter(N, F, E):
    e_per_w = E // NW
    nch = e_per_w // CHUNK
    rps = (N // NS) & ~7
    rem = N - rps * NS
    mesh = plsc.VectorSubcoreMesh(core_axis_name="c", subcore_axis_name="s")

    @functools.partial(
        pl.kernel,
        out_type=jax.ShapeDtypeStruct((NC, N, F), jnp.float32),
        mesh=mesh,
        scratch_types=[
            pltpu.VMEM((nch, CHUNK), jnp.int32),
            pltpu.VMEM((nch, CHUNK), jnp.int32),
            pltpu.VMEM((CHUNK, F), jnp.float32),
            pltpu.VMEM_SHARED((N, F), jnp.float32),
            pltpu.SemaphoreType.DMA,
        ],
    )
    def sc_scatter(ht_hbm, src_hbm, dst_hbm, zeros_hbm, out_hbm,
                   src_v, dst_v, rows_v, acc_sh, sem):
        c = lax.axis_index("c")
        s = lax.axis_index("s")
        wid = s * NC + c
        pltpu.sync_copy(src_hbm.at[wid], src_v)
        pltpu.sync_copy(dst_hbm.at[wid], dst_v)
        pltpu.sync_copy(zeros_hbm.at[pl.ds(s * rps, rps)],
                        acc_sh.at[pl.ds(s * rps, rps)])
        if rem:
            @pl.when(s == NS - 1)
            def _():
                pltpu.sync_copy(zeros_hbm.at[pl.ds(rps * NS, rem)],
                                acc_sh.at[pl.ds(rps * NS, rem)])
        plsc.subcore_barrier()

        @pl.loop(0, nch)
        def _(j):
            pltpu.async_copy(ht_hbm.at[src_v.at[j]], rows_v, sem).wait()
            pltpu.sync_copy(rows_v, acc_sh.at[dst_v.at[j]], add=True)

        plsc.subcore_barrier()
        pltpu.sync_copy(acc_sh.at[pl.ds(s * rps, rps)],
                        out_hbm.at[c, pl.ds(s * rps, rps)])
        if rem:
            @pl.when(s == NS - 1)
            def _():
                pltpu.sync_copy(acc_sh.at[pl.ds(rps * NS, rem)],
                                out_hbm.at[c, pl.ds(rps * NS, rem)])

    return sc_scatter


def kernel(h, edge_index, group_labels, W_self, W_groups):
    N, F = h.shape
    E = edge_index.shape[1]
    e_per_w = E // NW
    nch = e_per_w // CHUNK

    ht = pl.pallas_call(
        _group_transform_body,
        out_shape=jax.ShapeDtypeStruct((N, F), jnp.float32),
    )(h, group_labels.reshape(N, 1), W_groups)

    src = edge_index[0].reshape(NW, nch, CHUNK)
    dst = edge_index[1].reshape(NW, nch, CHUNK)
    zeros = jnp.zeros((N, F), jnp.float32)
    partials = _make_sc_scatter(N, F, E)(ht, src, dst, zeros)

    out = pl.pallas_call(
        _final_body,
        out_shape=jax.ShapeDtypeStruct((N, F), jnp.float32),
    )(h, W_self, partials)
    return out
